# baseline (speedup 1.0000x reference)
import functools

import jax
import jax.numpy as jnp
import numpy as np
from jax import lax
from jax.experimental import pallas as pl
from jax.experimental.pallas import tpu as pltpu
from jax.experimental.pallas import tpu_sc as plsc

B = 16384
F = 26
V = 1000000
MAX_DIM = 16
TEMP = 5.0
FS_WEIGHT = 1.0

NC = 2
NS = 16
NW = NC * NS
BPW = B // NW
NJ = BPW // 128

_CONST = {}


def _threefry2x32(k0, k1, x0, x1):
    ks = [np.uint32(k0), np.uint32(k1),
          np.uint32(np.uint32(k0) ^ np.uint32(k1) ^ np.uint32(0x1BD11BDA))]
    rotations = [[13, 15, 26, 6], [17, 29, 16, 24]]
    x0 = (x0 + ks[0]).astype(np.uint32)
    x1 = (x1 + ks[1]).astype(np.uint32)
    for i in range(5):
        for r in rotations[i % 2]:
            x0 = (x0 + x1).astype(np.uint32)
            x1 = ((x1 << np.uint32(r)) | (x1 >> np.uint32(32 - r))).astype(np.uint32)
            x1 = (x1 ^ x0).astype(np.uint32)
        x0 = (x0 + ks[(i + 1) % 3]).astype(np.uint32)
        x1 = (x1 + ks[(i + 2) % 3] + np.uint32(i + 1)).astype(np.uint32)
    return x0, x1


def _consts():
    if "p2" not in _CONST:
        size = F * B
        b1, b2 = _threefry2x32(0, 42, np.zeros(size, np.uint32),
                               np.arange(size, dtype=np.uint32))
        bits = b1 ^ b2
        u = (((bits >> np.uint32(9)) | np.uint32(0x3F800000))
             .view(np.float32) - np.float32(1.0)).reshape(F, B)
        perm = np.argsort(u, axis=1, kind="stable").astype(np.int64)
        p2 = (perm * F + np.arange(F, dtype=np.int64)[:, None]).astype(np.int32)
        p2 = p2.reshape(F, NW, BPW).transpose(1, 0, 2).reshape(NW, F * BPW).copy()
        _CONST["p2"] = p2
    return _CONST["p2"]


_P2 = _consts()


def _body(tflat, inflat, idxr, th, wt, p2r, out, fs,
          p2v, iv, iv2, x1v, x2v, outv, thv, wv, av, cv, s_i2, s_x1, s_x2):
    cid = lax.axis_index("c")
    sid = lax.axis_index("s")
    w = sid * NC + cid

    pltpu.sync_copy(p2r.at[w], p2v)
    pltpu.async_copy(inflat.at[p2v], iv2, s_i2)

    pltpu.sync_copy(idxr.at[w], iv)
    pltpu.sync_copy(th, thv)
    pltpu.sync_copy(wt, wv)

    def _off1(f, c):
        offv = jnp.full((16,), f * V, jnp.int32)
        for r in range(BPW // 16):
            sl = pl.ds(f * BPW + r * 16, 16)
            iv[sl] = iv[sl] + offv
        return c
    lax.fori_loop(0, F, _off1, 0, unroll=False)

    pltpu.async_copy(tflat.at[iv], x1v, s_x1)

    pltpu.make_async_copy(inflat.at[p2v], iv2, s_i2).wait()

    def _off2(f, c):
        offv = jnp.full((16,), f * V, jnp.int32)
        for r in range(BPW // 16):
            sl = pl.ds(f * BPW + r * 16, 16)
            iv2[sl] = iv2[sl] + offv
        return c
    lax.fori_loop(0, F, _off2, 0, unroll=False)

    pltpu.async_copy(tflat.at[iv2], x2v, s_x2)

    zv = jnp.zeros((16,), jnp.float32)
    for j in range(NJ):
        for l in range(8):
            outv[j, pl.ds(l * 16, 16)] = zv

    tr0 = thv[pl.ds(0, 16)]
    tr1 = thv[pl.ds(16, 16)]
    wr0 = wv[pl.ds(0, 16)]
    wr1 = wv[pl.ds(16, 16)]
    g0 = 1.0 / (1.0 + jnp.exp(tr0 * (-TEMP)))
    g1 = 1.0 / (1.0 + jnp.exp(tr1 * (-TEMP)))
    a0 = g0 * wr0
    c0 = (1.0 - g0) * wr0
    a1 = g1 * wr1
    c1 = (1.0 - g1) * wr1
    for f in range(F):
        asrc = a0 if f < 16 else a1
        csrc = c0 if f < 16 else c1
        lane = f % 16
        av[pl.ds(f * 16, 16)] = jnp.full((16,), asrc[lane])
        cv[pl.ds(f * 16, 16)] = jnp.full((16,), csrc[lane])

    pltpu.make_async_copy(tflat.at[iv], x1v, s_x1).wait()
    pltpu.make_async_copy(tflat.at[iv2], x2v, s_x2).wait()

    def _acc(f, c):
        a = av[pl.ds(f * 16, 16)]
        cc = cv[pl.ds(f * 16, 16)]
        for j in range(NJ):
            for l in range(8):
                sl = pl.ds(f * BPW + j * 128 + l * 16, 16)
                o = (j, pl.ds(l * 16, 16))
                outv[o] = outv[o] + a * x1v[sl] + cc * x2v[sl]
        return c
    lax.fori_loop(0, F, _acc, 0, unroll=False)

    pltpu.sync_copy(outv, out.at[w])

    @pl.when(w == 0)
    def _fs():
        s = g0[0]
        for i in range(1, 16):
            s = s + g0[i]
        for i in range(F - 16):
            s = s + g1[i]
        outv[0, pl.ds(0, 16)] = jnp.full((16,), s * (FS_WEIGHT * 0.1 / F))
        pltpu.sync_copy(outv.at[0, pl.ds(0, 16)], fs)


@functools.partial(
    pl.kernel,
    out_type=[
        jax.ShapeDtypeStruct((NW, NJ, 128), jnp.float32),
        jax.ShapeDtypeStruct((16,), jnp.float32),
    ],
    mesh=plsc.VectorSubcoreMesh(core_axis_name="c", subcore_axis_name="s"),
    scratch_types=[
        pltpu.VMEM((F * BPW,), jnp.int32),
        pltpu.VMEM((F * BPW,), jnp.int32),
        pltpu.VMEM((F * BPW,), jnp.int32),
        pltpu.VMEM((F * BPW,), jnp.float32),
        pltpu.VMEM((F * BPW,), jnp.float32),
        pltpu.VMEM((NJ, 128), jnp.float32),
        pltpu.VMEM((32,), jnp.float32),
        pltpu.VMEM((32,), jnp.float32),
        pltpu.VMEM((F * 16,), jnp.float32),
        pltpu.VMEM((F * 16,), jnp.float32),
        pltpu.SemaphoreType.DMA,
        pltpu.SemaphoreType.DMA,
        pltpu.SemaphoreType.DMA,
    ],
)
def _sc_call(tflat, inflat, idxr, th, wt, p2r, out, fs, *scratch):
    _body(tflat, inflat, idxr, th, wt, p2r, out, fs, *scratch)


def kernel(inputs, tables, theta, weight):
    p2r = jnp.asarray(_P2)
    tflat = tables.reshape(F * V)
    inflat = inputs.reshape(B * F)
    idxr = inputs.T.reshape(F, NW, BPW).transpose(1, 0, 2).reshape(NW, F * BPW)
    thc = jnp.pad(theta[:, 0], (0, 32 - F))
    wc = jnp.pad(weight[:, 0], (0, 32 - F))
    out, fs = _sc_call(tflat, inflat, idxr, thc, wc, p2r)
    return out.reshape(B), fs[0]
